# baseline (speedup 1.0000x reference)
import functools

import jax
import jax.numpy as jnp
import numpy as np
from jax import lax
from jax.experimental import pallas as pl
from jax.experimental.pallas import tpu as pltpu
from jax.experimental.pallas import tpu_sc as plsc

_B, _N, _D, _C, _K, _O = 16, 512, 25, 8, 32, 8
_ROW = _D * _O
_LCOL = 256
_WORDS = _LCOL // 2
_NODES = _B * _N
_NPC = 4
_CHUNK_IDX = _NPC * _K
_NWORK = 32
_CPW = _NODES // _NPC // _NWORK
_NPW = _NODES // _NWORK
_VB = 7
_OCOL = 32 * _VB


def _tc_weight_body(x_ref, w_ref, out_ref):
    m = jnp.dot(
        x_ref[...], w_ref[...], preferred_element_type=jnp.float32
    ).astype(jnp.bfloat16)
    lo = m[:, :_WORDS]
    hi = m[:, _WORDS:]
    lo32 = lax.bitcast_convert_type(lo, jnp.uint16).astype(jnp.uint32)
    hi32 = lax.bitcast_convert_type(hi, jnp.uint16).astype(jnp.uint32)
    word = lo32 | (hi32 << jnp.uint32(16))
    out_ref[...] = lax.bitcast_convert_type(word, jnp.int32)


def _sc_gather_body(y_hbm, idx_hbm, out_hbm, idx_v, g0, g1, obuf, s0, s1):
    nc = plsc.get_sparse_core_info().num_cores
    wid = lax.axis_index("s") * nc + lax.axis_index("c")
    base_chunk = wid * _CPW

    pltpu.sync_copy(idx_hbm.at[pl.ds(base_chunk, _CPW)], idx_v)

    def start(ci, gb, sem):
        pltpu.make_async_copy(y_hbm.at[idx_v.at[ci]], gb, sem).start()

    def wait(gb, sem):
        pltpu.make_async_copy(y_hbm.at[idx_v.at[0]], gb, sem).wait()

    def process(gb, ci):
        for t in range(_NPC):
            def r_body(r, acc):
                new = []
                for v in range(_VB):
                    wvec = gb[t * _K + r, pl.ds(v * 16, 16)]
                    a = lax.bitcast_convert_type(
                        wvec << jnp.int32(16), jnp.float32
                    )
                    b = lax.bitcast_convert_type(
                        wvec & jnp.int32(-65536), jnp.float32
                    )
                    new.append(acc[2 * v] + a)
                    new.append(acc[2 * v + 1] + b)
                return tuple(new)

            acc0 = tuple(jnp.zeros((16,), jnp.float32) for _ in range(2 * _VB))
            acc = lax.fori_loop(0, _K, r_body, acc0)
            for v in range(_VB):
                obuf[ci * _NPC + t, pl.ds(v * 32, 16)] = acc[2 * v]
                obuf[ci * _NPC + t, pl.ds(v * 32 + 16, 16)] = acc[2 * v + 1]

    start(0, g0, s0)

    def pair_body(p, carry):
        start(2 * p + 1, g1, s1)
        wait(g0, s0)
        process(g0, 2 * p)

        @pl.when(p < _CPW // 2 - 1)
        def _():
            start(2 * p + 2, g0, s0)

        wait(g1, s1)
        process(g1, 2 * p + 1)
        return carry

    lax.fori_loop(0, _CPW // 2, pair_body, 0)
    pltpu.sync_copy(obuf, out_hbm.at[pl.ds(wid * _NPW, _NPW)])


def kernel(x, knn_matrix, w, conv_w, conv_b):
    del conv_w, conv_b
    B, N, D, C = x.shape
    k = knn_matrix.shape[-1]
    out_ch = w.shape[-1]

    w2 = w.reshape(C * k, out_ch)
    nw = (w2 ** 2) / jnp.sum(w2, axis=0)
    nw3 = nw.reshape(C, k, out_ch)
    eye = jnp.eye(D, dtype=jnp.float32)
    wbig = jnp.einsum("de,cfo->fdceo", eye, nw3).reshape(k, D * C, D * out_ch)
    wbig = jnp.pad(wbig, ((0, 0), (0, 0), (0, _LCOL - _ROW)))
    word_pos = np.arange(_WORDS)
    u, i = word_pos // 16, word_pos % 16
    lo_cols = 32 * u + i
    hi_cols = 32 * u + 16 + i
    cols = np.concatenate([lo_cols, hi_cols])
    wfull = wbig[:, :, cols].reshape(k * D * C, _LCOL).astype(jnp.bfloat16)

    x2 = x.reshape(B * N, D * C).astype(jnp.bfloat16)

    n_jblk = 4
    jb_rows = B * N // n_jblk
    y = pl.pallas_call(
        _tc_weight_body,
        grid=(n_jblk, k),
        in_specs=[
            pl.BlockSpec((jb_rows, D * C), lambda jb, f: (jb, 0)),
            pl.BlockSpec((D * C, _LCOL), lambda jb, f: (f, 0)),
        ],
        out_specs=pl.BlockSpec(
            (jb_rows, _WORDS), lambda jb, f: (f * n_jblk + jb, 0)
        ),
        out_shape=jax.ShapeDtypeStruct((k * B * N, _WORDS), jnp.int32),
        compiler_params=pltpu.CompilerParams(
            dimension_semantics=("parallel", "arbitrary"),
        ),
    )(x2, wfull)

    knn_i = knn_matrix.astype(jnp.int32)
    boff = (jnp.arange(B, dtype=jnp.int32) * N).reshape(B, 1, 1)
    foff = (jnp.arange(k, dtype=jnp.int32) * (B * N)).reshape(1, 1, k)
    idx_all = (knn_i + boff + foff).reshape(_NODES * k // _CHUNK_IDX, _CHUNK_IDX)

    mesh = plsc.VectorSubcoreMesh(core_axis_name="c", subcore_axis_name="s")
    sc_fn = functools.partial(
        pl.kernel,
        mesh=mesh,
        out_type=jax.ShapeDtypeStruct((_NODES, _OCOL), jnp.float32),
        scratch_types=[
            pltpu.VMEM((_CPW, _CHUNK_IDX), jnp.int32),
            pltpu.VMEM((_CHUNK_IDX, _WORDS), jnp.int32),
            pltpu.VMEM((_CHUNK_IDX, _WORDS), jnp.int32),
            pltpu.VMEM((_NPW, _OCOL), jnp.float32),
            pltpu.SemaphoreType.DMA,
            pltpu.SemaphoreType.DMA,
        ],
    )(_sc_gather_body)
    out2 = sc_fn(y, idx_all)

    return out2[:, :_ROW].reshape(B, N, D, out_ch)


# speedup vs baseline: 1.0292x; 1.0292x over previous
import functools

import jax
import jax.numpy as jnp
import numpy as np
from jax import lax
from jax.experimental import pallas as pl
from jax.experimental.pallas import tpu as pltpu
from jax.experimental.pallas import tpu_sc as plsc

_B, _N, _D, _C, _K, _O = 16, 512, 25, 8, 32, 8
_ROW = _D * _O
_LCOL = 256
_WORDS = _LCOL // 2
_NODES = _B * _N
_NPC = 4
_CHUNK_IDX = _NPC * _K
_NWORK = 32
_CPW = _NODES // _NPC // _NWORK
_NPW = _NODES // _NWORK
_VB = 7
_OCOL = 32 * _VB
_KH = _K // 2
_NPCH = _CHUNK_IDX // _KH
_CPWH = _NODES // _NPCH // _NWORK


def _tc_weight_body(x_ref, w_ref, out_ref):
    m = jnp.dot(
        x_ref[...], w_ref[...], preferred_element_type=jnp.float32
    ).astype(jnp.bfloat16)
    lo = m[:, :_WORDS]
    hi = m[:, _WORDS:]
    lo32 = lax.bitcast_convert_type(lo, jnp.uint16).astype(jnp.uint32)
    hi32 = lax.bitcast_convert_type(hi, jnp.uint16).astype(jnp.uint32)
    word = lo32 | (hi32 << jnp.uint32(16))
    out_ref[...] = lax.bitcast_convert_type(word, jnp.int32)


def _sc_gather_body(y_hbm, idx_hbm, out_hbm, idx_v, g0, g1, obuf, s0, s1):
    nc = plsc.get_sparse_core_info().num_cores
    wid = lax.axis_index("s") * nc + lax.axis_index("c")
    base_chunk = wid * _CPWH

    pltpu.sync_copy(idx_hbm.at[pl.ds(base_chunk, _CPWH)], idx_v)

    def start(ci, gb, sem):
        pltpu.make_async_copy(y_hbm.at[idx_v.at[ci]], gb, sem).start()

    def wait(gb, sem):
        pltpu.make_async_copy(y_hbm.at[idx_v.at[0]], gb, sem).wait()

    def process(gb, ci):
        for t in range(_NPCH):
            def r_body(r, acc):
                new = []
                for v in range(_VB):
                    wvec = gb[t * _KH + r, pl.ds(v * 16, 16)]
                    a = lax.bitcast_convert_type(
                        wvec << jnp.int32(16), jnp.float32
                    )
                    b = lax.bitcast_convert_type(
                        wvec & jnp.int32(-65536), jnp.float32
                    )
                    new.append(acc[2 * v] + a)
                    new.append(acc[2 * v + 1] + b)
                return tuple(new)

            acc0 = tuple(jnp.zeros((16,), jnp.float32) for _ in range(2 * _VB))
            acc = lax.fori_loop(0, _KH, r_body, acc0)
            for v in range(_VB):
                obuf[ci * _NPCH + t, pl.ds(v * 32, 16)] = acc[2 * v]
                obuf[ci * _NPCH + t, pl.ds(v * 32 + 16, 16)] = acc[2 * v + 1]

    start(0, g0, s0)

    def pair_body(p, carry):
        start(2 * p + 1, g1, s1)
        wait(g0, s0)
        process(g0, 2 * p)

        @pl.when(p < _CPWH // 2 - 1)
        def _():
            start(2 * p + 2, g0, s0)

        wait(g1, s1)
        process(g1, 2 * p + 1)
        return carry

    lax.fori_loop(0, _CPWH // 2, pair_body, 0)
    pltpu.sync_copy(obuf, out_hbm.at[pl.ds(wid * _NPW, _NPW)])


def kernel(x, knn_matrix, w, conv_w, conv_b):
    del conv_w, conv_b
    B, N, D, C = x.shape
    k = knn_matrix.shape[-1]
    out_ch = w.shape[-1]

    w2 = w.reshape(C * k, out_ch)
    nw = (w2 ** 2) / jnp.sum(w2, axis=0)
    nw3 = nw.reshape(C, k, out_ch)
    eye = jnp.eye(D, dtype=jnp.float32)
    wbig = jnp.einsum("de,cfo->fdceo", eye, nw3).reshape(k, D * C, D * out_ch)
    wbig = jnp.pad(wbig, ((0, 0), (0, 0), (0, _LCOL - _ROW)))
    word_pos = np.arange(_WORDS)
    u, i = word_pos // 16, word_pos % 16
    lo_cols = 32 * u + i
    hi_cols = 32 * u + 16 + i
    cols = np.concatenate([lo_cols, hi_cols])
    wfull = wbig[:, :, cols].reshape(k * D * C, _LCOL).astype(jnp.bfloat16)

    x2 = x.reshape(B * N, D * C).astype(jnp.bfloat16)

    n_jblk = 4
    jb_rows = B * N // n_jblk
    kh = k // 2
    knn_i = knn_matrix.astype(jnp.int32)
    boff = (jnp.arange(B, dtype=jnp.int32) * N).reshape(B, 1, 1)
    foff = (jnp.arange(kh, dtype=jnp.int32) * (B * N)).reshape(1, 1, kh)
    mesh = plsc.VectorSubcoreMesh(core_axis_name="c", subcore_axis_name="s")
    sc_fn = functools.partial(
        pl.kernel,
        mesh=mesh,
        out_type=jax.ShapeDtypeStruct((_NODES, _OCOL), jnp.float32),
        scratch_types=[
            pltpu.VMEM((_CPWH, _CHUNK_IDX), jnp.int32),
            pltpu.VMEM((_CHUNK_IDX, _WORDS), jnp.int32),
            pltpu.VMEM((_CHUNK_IDX, _WORDS), jnp.int32),
            pltpu.VMEM((_NPW, _OCOL), jnp.float32),
            pltpu.SemaphoreType.DMA,
            pltpu.SemaphoreType.DMA,
        ],
    )(_sc_gather_body)
    wfull3 = wfull.reshape(k, D * C, _LCOL)

    outs = []
    for h in range(2):
        y_h = pl.pallas_call(
            _tc_weight_body,
            grid=(n_jblk, kh),
            in_specs=[
                pl.BlockSpec((jb_rows, D * C), lambda jb, f: (jb, 0)),
                pl.BlockSpec((D * C, _LCOL), lambda jb, f: (f, 0)),
            ],
            out_specs=pl.BlockSpec(
                (jb_rows, _WORDS), lambda jb, f: (f * n_jblk + jb, 0)
            ),
            out_shape=jax.ShapeDtypeStruct((kh * B * N, _WORDS), jnp.int32),
            compiler_params=pltpu.CompilerParams(
                dimension_semantics=("parallel", "arbitrary"),
            ),
        )(x2, wfull3[h * kh:(h + 1) * kh].reshape(kh * D * C, _LCOL))

        idx_h = (knn_i[:, :, h * kh:(h + 1) * kh] + boff + foff).reshape(
            _NODES * kh // _CHUNK_IDX, _CHUNK_IDX
        )
        outs.append(sc_fn(y_h, idx_h))

    out2 = outs[0] + outs[1]
    return out2[:, :_ROW].reshape(B, N, D, out_ch)
